# baseline (speedup 1.0000x reference)
import functools

import jax
import jax.numpy as jnp
from jax import lax
from jax.experimental import pallas as pl
from jax.experimental.pallas import tpu as pltpu
from jax.experimental.pallas import tpu_sc as plsc

_VOCAB = 1000000
_EMBED = 64
_B = 16384
_L = 50
_N = _B * _L

_NC = 2
_NS = 16
_NW = _NC * _NS
_KC = 128

_LH = _L // 2
_NH = _B * _LH
_PER_W = _NH // _NW
_NCHUNK = _PER_W // _KC
_NPAIR = _NCHUNK // 2
_NBUF = 5
_PROWS_W = _PER_W // 2


def _gather_body(emb_hbm, idx_hbm, out_hbm, idx_v, bufs, gls, grs):
    wid = lax.axis_index("s") * _NC + lax.axis_index("c")
    pbase = wid * _PROWS_W
    pltpu.sync_copy(idx_hbm.at[wid], idx_v)

    def _glo(t, buf, sem):
        return pltpu.make_async_copy(
            emb_hbm.at[idx_v.at[2 * t]], buf.at[0], sem)

    def _ghi(t, buf, sem):
        return pltpu.make_async_copy(
            emb_hbm.at[idx_v.at[2 * t + 1]], buf.at[1], sem)

    def _wlo(buf, sem, t):
        return pltpu.make_async_copy(
            buf.at[0], out_hbm.at[pl.ds(pbase + t * _KC, _KC), 0:_EMBED], sem)

    def _whi(buf, sem, t):
        return pltpu.make_async_copy(
            buf.at[1], out_hbm.at[pl.ds(pbase + t * _KC, _KC), _EMBED:128], sem)

    for i in range(_NBUF):
        _glo(i, bufs[i], gls[i]).start()
        _ghi(i, bufs[i], grs[i]).start()

    def body(g, carry):
        t0 = _NBUF * g
        for i in range(_NBUF):
            t = t0 + i
            _glo(t, bufs[i], gls[i]).wait()
            _wlo(bufs[i], gls[i], t).start()
            _ghi(t, bufs[i], grs[i]).wait()
            _whi(bufs[i], grs[i], t).start()

        @pl.when(t0 + _NBUF < _NPAIR)
        def _():
            for i in range(_NBUF):
                t = t0 + i
                _wlo(bufs[i], gls[i], t).wait()
                _glo(t + _NBUF, bufs[i], gls[i]).start()
                _whi(bufs[i], grs[i], t).wait()
                _ghi(t + _NBUF, bufs[i], grs[i]).start()

        return carry

    lax.fori_loop(0, _NPAIR // _NBUF, body, 0)
    for i in range(_NBUF):
        t = _NPAIR - _NBUF + i
        _wlo(bufs[i], gls[i], t).wait()
        _whi(bufs[i], grs[i], t).wait()


def _gather_entry(emb_hbm, idx_hbm, out_hbm, idx_v, *sems):
    bufs = sems[:_NBUF]
    gls = sems[_NBUF:2 * _NBUF]
    grs = sems[2 * _NBUF:]
    _gather_body(emb_hbm, idx_hbm, out_hbm, idx_v, bufs, gls, grs)


_gather = functools.partial(
    pl.kernel,
    out_type=jax.ShapeDtypeStruct((_NH // 2, 128), jnp.float32),
    mesh=plsc.VectorSubcoreMesh(core_axis_name="c", subcore_axis_name="s"),
    scratch_types=(
        [pltpu.VMEM((_NCHUNK, _KC), jnp.int32)]
        + [pltpu.VMEM((2, _KC, _EMBED), jnp.float32)] * _NBUF
        + [pltpu.SemaphoreType.DMA] * (2 * _NBUF)
    ),
    compiler_params=pltpu.CompilerParams(use_tc_tiling_on_sc=False),
)(_gather_entry)


_HB = _B // 2


def _linear_body(h_ref, w_ref, b_ref, o_ref):
    h = h_ref[...]
    w = w_ref[...]
    bb = b_ref[...]
    nt = (((1,), (1,)), ((), ()))
    lo = lax.dot_general(w, h[:, :_EMBED], nt,
                         preferred_element_type=jnp.float32)
    hi = lax.dot_general(w, h[:, _EMBED:], nt,
                         preferred_element_type=jnp.float32)
    o_ref[0, :, :_HB] = lo + bb
    o_ref[0, :, _HB:] = hi + bb


def _linear_body2(h_ref, w_ref, b_ref, y_ref, o_ref):
    _linear_body(h_ref, w_ref, b_ref, o_ref)


def _linear1(hp, w, b1):
    return pl.pallas_call(
        _linear_body,
        grid=(_LH,),
        in_specs=[
            pl.BlockSpec((_HB, 128), lambda i: (i, 0)),
            pl.BlockSpec((_EMBED, _EMBED), lambda i: (0, 0)),
            pl.BlockSpec((_EMBED, 1), lambda i: (0, 0)),
        ],
        out_specs=pl.BlockSpec((1, _EMBED, _B), lambda i: (i, 0, 0)),
        out_shape=jax.ShapeDtypeStruct((_L, _EMBED, _B), jnp.float32),
    )(hp, w, b1)


def _linear2(hp, w, b1, y):
    return pl.pallas_call(
        _linear_body2,
        grid=(_LH,),
        in_specs=[
            pl.BlockSpec((_HB, 128), lambda i: (i, 0)),
            pl.BlockSpec((_EMBED, _EMBED), lambda i: (0, 0)),
            pl.BlockSpec((_EMBED, 1), lambda i: (0, 0)),
            pl.BlockSpec(memory_space=pl.ANY),
        ],
        out_specs=pl.BlockSpec((1, _EMBED, _B), lambda i: (i + _LH, 0, 0)),
        out_shape=jax.ShapeDtypeStruct((_L, _EMBED, _B), jnp.float32),
        input_output_aliases={3: 0},
    )(hp, w, b1, y)


def kernel(x, emb, W, b):
    xi = jnp.transpose(
        x.astype(jnp.int32).T.reshape(_L, 2, _HB // _KC, _KC), (0, 2, 1, 3)
    ).reshape(2, _NW, _NCHUNK, _KC)
    b1 = b.reshape(_EMBED, 1)
    h1 = _gather(emb, xi[0])
    h2 = _gather(emb, xi[1])
    y1 = _linear1(h1, W, b1)
    y2 = _linear2(h2, W, b1, y1)
    return jnp.transpose(y2, (2, 0, 1))


# speedup vs baseline: 1.0060x; 1.0060x over previous
import functools

import jax
import jax.numpy as jnp
from jax import lax
from jax.experimental import pallas as pl
from jax.experimental.pallas import tpu as pltpu
from jax.experimental.pallas import tpu_sc as plsc

_VOCAB = 1000000
_EMBED = 64
_B = 16384
_L = 50
_N = _B * _L

_NC = 2
_NS = 16
_NW = _NC * _NS
_KC = 128
_PER_W = _N // _NW
_NCHUNK = _PER_W // _KC


_NPAIR = _NCHUNK // 2
_NBUF = 5
_PROWS_W = _PER_W // 2


def _gather_body(emb_hbm, idx_hbm, out_hbm, idx_v, bufs, gls, grs):
    wid = lax.axis_index("s") * _NC + lax.axis_index("c")
    pbase = wid * _PROWS_W
    pltpu.sync_copy(idx_hbm.at[wid], idx_v)

    def _glo(t, buf, sem):
        return pltpu.make_async_copy(
            emb_hbm.at[idx_v.at[2 * t]], buf.at[0], sem)

    def _ghi(t, buf, sem):
        return pltpu.make_async_copy(
            emb_hbm.at[idx_v.at[2 * t + 1]], buf.at[1], sem)

    def _wlo(buf, sem, t):
        return pltpu.make_async_copy(
            buf.at[0], out_hbm.at[pl.ds(pbase + t * _KC, _KC), 0:_EMBED], sem)

    def _whi(buf, sem, t):
        return pltpu.make_async_copy(
            buf.at[1], out_hbm.at[pl.ds(pbase + t * _KC, _KC), _EMBED:128], sem)

    for i in range(_NBUF):
        _glo(i, bufs[i], gls[i]).start()
        _ghi(i, bufs[i], grs[i]).start()

    def body(g, carry):
        t0 = _NBUF * g
        for i in range(_NBUF):
            t = t0 + i
            _glo(t, bufs[i], gls[i]).wait()
            _wlo(bufs[i], gls[i], t).start()
            _ghi(t, bufs[i], grs[i]).wait()
            _whi(bufs[i], grs[i], t).start()

        @pl.when(t0 + _NBUF < _NPAIR)
        def _():
            for i in range(_NBUF):
                t = t0 + i
                _wlo(bufs[i], gls[i], t).wait()
                _glo(t + _NBUF, bufs[i], gls[i]).start()
                _whi(bufs[i], grs[i], t).wait()
                _ghi(t + _NBUF, bufs[i], grs[i]).start()

        return carry

    lax.fori_loop(0, _NPAIR // _NBUF, body, 0)
    for i in range(_NBUF):
        t = _NPAIR - _NBUF + i
        _wlo(bufs[i], gls[i], t).wait()
        _whi(bufs[i], grs[i], t).wait()


def _gather_entry(emb_hbm, idx_hbm, out_hbm, idx_v, *sems):
    bufs = sems[:_NBUF]
    gls = sems[_NBUF:2 * _NBUF]
    grs = sems[2 * _NBUF:]
    _gather_body(emb_hbm, idx_hbm, out_hbm, idx_v, bufs, gls, grs)


_gather = functools.partial(
    pl.kernel,
    out_type=jax.ShapeDtypeStruct((_N // 2, 128), jnp.float32),
    mesh=plsc.VectorSubcoreMesh(core_axis_name="c", subcore_axis_name="s"),
    scratch_types=(
        [pltpu.VMEM((_NCHUNK, _KC), jnp.int32)]
        + [pltpu.VMEM((2, _KC, _EMBED), jnp.float32)] * _NBUF
        + [pltpu.SemaphoreType.DMA] * (2 * _NBUF)
    ),
    compiler_params=pltpu.CompilerParams(use_tc_tiling_on_sc=False),
)(_gather_entry)


_HB = _B // 2


def _linear_body(h_ref, w_ref, b_ref, o_ref):
    h = h_ref[...]
    w = w_ref[...]
    bb = b_ref[...]
    nt = (((1,), (1,)), ((), ()))
    lo = lax.dot_general(w, h[:, :_EMBED], nt,
                         preferred_element_type=jnp.float32)
    hi = lax.dot_general(w, h[:, _EMBED:], nt,
                         preferred_element_type=jnp.float32)
    o_ref[0, :, :_HB] = lo + bb
    o_ref[0, :, _HB:] = hi + bb


def _linear(hp, w, b1):
    return pl.pallas_call(
        _linear_body,
        grid=(_L,),
        in_specs=[
            pl.BlockSpec((_HB, 128), lambda i: (i, 0)),
            pl.BlockSpec((_EMBED, _EMBED), lambda i: (0, 0)),
            pl.BlockSpec((_EMBED, 1), lambda i: (0, 0)),
        ],
        out_specs=pl.BlockSpec((1, _EMBED, _B), lambda i: (i, 0, 0)),
        out_shape=jax.ShapeDtypeStruct((_L, _EMBED, _B), jnp.float32),
    )(hp, w, b1)


def kernel(x, emb, W, b):
    xi = jnp.transpose(
        x.astype(jnp.int32).T.reshape(_L, 2, _HB // _KC, _KC), (0, 2, 1, 3)
    ).reshape(-1)
    idx = xi.reshape(_NW, _NCHUNK, _KC)
    hp = _gather(emb, idx)
    ot = _linear(hp, W, b.reshape(_EMBED, 1))
    return jnp.transpose(ot, (2, 0, 1))
